# baseline (speedup 1.0000x reference)
import jax
import jax.numpy as jnp
from jax.experimental import pallas as pl

_N = 1024
_LATENT = 64
_HIDDEN = 256
_OUT = 2000
_NUM_OMICS = 3
_EPS = 1e-3
_ROWS = 8

_TDOT = (((0,), (0,)), ((), ()))


def _moum_kernel(a_ref, x_ref, w1_ref, b1_ref, w2_ref, b2_ref,
                 wg1_ref, bg1_ref, g1_ref, be1_ref,
                 wg2_ref, bg2_ref, g2_ref, be2_ref, out_ref):
    a = a_ref[...].astype(jnp.bfloat16)
    ones = jnp.ones((_N, 1), jnp.bfloat16)
    deg = jax.lax.dot_general(ones, a, _TDOT,
                              preferred_element_type=jnp.float32) + 1.0
    norm = jnp.transpose(jax.lax.rsqrt(jnp.maximum(deg, 1.0)))

    def tdot_f32(lhs_bf, rhs_f32):
        hi = rhs_f32.astype(jnp.bfloat16)
        lo = (rhs_f32 - hi.astype(jnp.float32)).astype(jnp.bfloat16)
        return (jax.lax.dot_general(lhs_bf, hi, _TDOT, preferred_element_type=jnp.float32)
                + jax.lax.dot_general(lhs_bf, lo, _TDOT, preferred_element_type=jnp.float32))

    x = x_ref[...]
    y = x * norm
    z = tdot_f32(a, y) + y
    agg = z * norm
    x1 = jnp.maximum(
        jnp.dot(agg, w1_ref[...], preferred_element_type=jnp.float32) + b1_ref[...],
        0.0)

    y1 = x1 * norm
    z2 = tdot_f32(a[:, 0:_ROWS], y1) + y1[0:_ROWS, :]
    agg2 = z2 * norm[0:_ROWS, :]
    x2 = jnp.maximum(
        jnp.dot(agg2, w2_ref[...], preferred_element_type=jnp.float32) + b2_ref[...],
        0.0)

    inv = 1.0 / jnp.sqrt(1.0 + _EPS)
    rows = []
    for i in range(_NUM_OMICS):
        xi = x2[i:i + 1, :]
        h = jnp.dot(xi, wg1_ref[i], preferred_element_type=jnp.float32) + bg1_ref[i:i + 1, :]
        h = g1_ref[i:i + 1, :] * h * inv + be1_ref[i:i + 1, :]
        h = jnp.maximum(h, 0.0)
        o = jnp.dot(h, wg2_ref[i], preferred_element_type=jnp.float32) + bg2_ref[i:i + 1, :]
        o = g2_ref[i:i + 1, :] * o * inv + be2_ref[i:i + 1, :]
        rows.append(o)
    out_ref[...] = jnp.concatenate(rows, axis=0)


def kernel(latent_vectors, adjacency_matrix, W_gnn1, b_gnn1, W_gnn2, b_gnn2,
           Wg1, bg1, gamma1, beta1, Wg2, bg2, gamma2, beta2):
    return pl.pallas_call(
        _moum_kernel,
        out_shape=jax.ShapeDtypeStruct((_NUM_OMICS, _OUT), jnp.float32),
    )(adjacency_matrix, latent_vectors,
      W_gnn1, b_gnn1.reshape(1, _LATENT), W_gnn2, b_gnn2.reshape(1, _LATENT),
      Wg1, bg1, gamma1, beta1, Wg2, bg2, gamma2, beta2)


# speedup vs baseline: 1.1459x; 1.1459x over previous
import jax
import jax.numpy as jnp
from jax.experimental import pallas as pl

_N = 1024
_LATENT = 64
_HIDDEN = 256
_OUT = 2000
_NUM_OMICS = 3
_EPS = 1e-3
_ROWS = 8

_TDOT = (((0,), (0,)), ((), ()))


def _moum_kernel(a_ref, x_ref, w1_ref, b1_ref, w2_ref, b2_ref,
                 wg1_ref, bg1_ref, g1_ref, be1_ref,
                 wg2_ref, bg2_ref, g2_ref, be2_ref, out_ref):
    a = a_ref[...].astype(jnp.bfloat16)
    ones = jnp.ones((_N, 1), jnp.bfloat16)
    deg = jax.lax.dot_general(ones, a, _TDOT,
                              preferred_element_type=jnp.float32) + 1.0
    norm = jnp.transpose(jax.lax.rsqrt(jnp.maximum(deg, 1.0)))

    def tdot_f32(lhs_bf, rhs_f32):
        hi = rhs_f32.astype(jnp.bfloat16)
        lo = (rhs_f32 - hi.astype(jnp.float32)).astype(jnp.bfloat16)
        return (jax.lax.dot_general(lhs_bf, hi, _TDOT, preferred_element_type=jnp.float32)
                + jax.lax.dot_general(lhs_bf, lo, _TDOT, preferred_element_type=jnp.float32))

    x = x_ref[...]
    y = x * norm
    z = tdot_f32(a, y) + y
    agg = z * norm
    x1 = jnp.maximum(
        jnp.dot(agg, w1_ref[...], preferred_element_type=jnp.float32) + b1_ref[...],
        0.0)

    y1 = x1 * norm
    z2 = tdot_f32(a[:, 0:_ROWS], y1) + y1[0:_ROWS, :]
    agg2 = z2 * norm[0:_ROWS, :]
    x2 = jnp.maximum(
        jnp.dot(agg2, w2_ref[...], preferred_element_type=jnp.float32) + b2_ref[...],
        0.0)

    inv = 1.0 / jnp.sqrt(1.0 + _EPS)
    rows = []
    for i in range(_NUM_OMICS):
        xi = x2[i:i + 1, :]
        h = jnp.dot(xi, wg1_ref[i], preferred_element_type=jnp.float32) + bg1_ref[i:i + 1, :]
        h = g1_ref[i:i + 1, :] * h * inv + be1_ref[i:i + 1, :]
        h = jnp.maximum(h, 0.0)
        o = jnp.dot(h, wg2_ref[i], preferred_element_type=jnp.float32) + bg2_ref[i:i + 1, :]
        o = g2_ref[i:i + 1, :] * o * inv + be2_ref[i:i + 1, :]
        rows.append(o)
    out_ref[...] = jnp.concatenate(rows, axis=0)


def kernel(latent_vectors, adjacency_matrix, W_gnn1, b_gnn1, W_gnn2, b_gnn2,
           Wg1, bg1, gamma1, beta1, Wg2, bg2, gamma2, beta2):
    return pl.pallas_call(
        _moum_kernel,
        out_shape=jax.ShapeDtypeStruct((_NUM_OMICS, _OUT), jnp.float32),
    )(adjacency_matrix.astype(jnp.int8), latent_vectors,
      W_gnn1, b_gnn1.reshape(1, _LATENT), W_gnn2, b_gnn2.reshape(1, _LATENT),
      Wg1, bg1, gamma1, beta1, Wg2, bg2, gamma2, beta2)
